# baseline (speedup 1.0000x reference)
import functools

import jax
import jax.numpy as jnp
from jax import lax
from jax.experimental import pallas as pl
from jax.experimental.pallas import tpu as pltpu
from jax.experimental.pallas import tpu_sc as plsc

B = 1024
P = B * (B - 1) // 2
NW = 32
CHUNK = P // NW
HALF = 8192
H2 = CHUNK - HALF
TWO_B_M1 = 2 * B - 1
GUARD = 16
BUFN = GUARD + CHUNK + 32
XPAD = 1040
MAGIC = 0x5F3759DF
INV_SQRT2 = 0.7071067811865476


def _row_offset(i):
    return (i * (TWO_B_M1 - i)) >> 1


def _row_of(kscal):
    k = jnp.full((16,), kscal, jnp.int32)
    hd = jnp.float32(2095104.5) - jnp.float32(4.0) * k.astype(jnp.float32)
    r = plsc.bitcast(
        jnp.int32(MAGIC) - (plsc.bitcast(hd, jnp.int32) >> 1), jnp.float32
    )
    r = r * (jnp.float32(1.5) - (jnp.float32(0.5) * hd) * r * r)
    i_f = jnp.float32(B - 0.5) - jnp.float32(INV_SQRT2) * (hd * r)
    i0 = i_f.astype(jnp.int32)
    i0 = jnp.where(_row_offset(i0) > k, i0 - 1, i0)
    i0 = jnp.where(_row_offset(i0) > k, i0 - 1, i0)
    i0 = jnp.where(_row_offset(i0) > k, i0 - 1, i0)
    i0 = jnp.where(_row_offset(i0 + 1) <= k, i0 + 1, i0)
    return jnp.max(i0)


def _pairs_body(x_hbm, x1_hbm, x2_hbm, x_v, o1_v, o2_v, sem_x, sem_o):
    wid = lax.axis_index("s") * 2 + lax.axis_index("c")
    base = wid * CHUNK
    cp_x = pltpu.make_async_copy(x_hbm, x_v.at[pl.ds(0, B)], sem_x)
    cp_x.start()
    cp_x.wait()

    def walk(F, C, ceil_start):
        i_init = _row_of(base + F)
        pos_init = _row_offset(i_init) - base

        def cond(st):
            return st[1] < C

        def rbody(st):
            i, pos = st
            lim = jnp.minimum(jnp.int32(B - 1) - i, C - pos)
            d = jnp.int32(F) - pos
            if ceil_start:
                off0 = jnp.maximum(0, ((d + 15) >> 4) << 4)
            else:
                off0 = jnp.maximum(0, (d >> 4) << 4)
            n = jnp.maximum(0, (lim - off0 + 15) >> 4)
            splat = plsc.load_gather(x_v, [jnp.full((16,), i, jnp.int32)])
            p0 = pos + off0 + GUARD
            j0 = i + 1 + off0

            def ibody(t, c):
                o1_v[pl.ds(p0 + t * 16, 16)] = splat
                o2_v[pl.ds(p0 + t * 16, 16)] = x_v[pl.ds(j0 + t * 16, 16)]
                return c

            lax.fori_loop(0, n, ibody, 0)
            return i + 1, pos + (jnp.int32(B - 1) - i)

        lax.while_loop(cond, rbody, (i_init, pos_init))

    walk(0, HALF + 16, False)
    cp1a = pltpu.make_async_copy(
        o1_v.at[pl.ds(GUARD, HALF)], x1_hbm.at[pl.ds(base, HALF)], sem_o
    )
    cp1b = pltpu.make_async_copy(
        o2_v.at[pl.ds(GUARD, HALF)], x2_hbm.at[pl.ds(base, HALF)], sem_o
    )
    cp1a.start()
    cp1b.start()
    walk(HALF, CHUNK, True)
    cp2a = pltpu.make_async_copy(
        o1_v.at[pl.ds(GUARD + HALF, H2)], x1_hbm.at[pl.ds(base + HALF, H2)], sem_o
    )
    cp2b = pltpu.make_async_copy(
        o2_v.at[pl.ds(GUARD + HALF, H2)], x2_hbm.at[pl.ds(base + HALF, H2)], sem_o
    )
    cp2a.start()
    cp2b.start()
    cp1a.wait()
    cp1b.wait()
    cp2a.wait()
    cp2b.wait()


@functools.cache
def _build():
    return functools.partial(
        pl.kernel,
        out_type=(
            jax.ShapeDtypeStruct((P,), jnp.float32),
            jax.ShapeDtypeStruct((P,), jnp.float32),
        ),
        mesh=plsc.VectorSubcoreMesh(
            core_axis_name="c", subcore_axis_name="s", num_cores=2, num_subcores=16
        ),
        scratch_types=[
            pltpu.VMEM((XPAD,), jnp.float32),
            pltpu.VMEM((BUFN,), jnp.float32),
            pltpu.VMEM((BUFN,), jnp.float32),
            pltpu.SemaphoreType.DMA,
            pltpu.SemaphoreType.DMA,
        ],
        compiler_params=pltpu.CompilerParams(needs_layout_passes=False, disable_bounds_checks=True),
    )(_pairs_body)


def kernel(x):
    return _build()(x)


# speedup vs baseline: 1.0623x; 1.0623x over previous
import functools

import jax
import jax.numpy as jnp
from jax import lax
from jax.experimental import pallas as pl
from jax.experimental.pallas import tpu as pltpu
from jax.experimental.pallas import tpu_sc as plsc

B = 1024
P = B * (B - 1) // 2
NW = 32
CHUNK = P // NW
BUFN = CHUNK + 32
XPAD = 1040
TWO_B_M1 = 2 * B - 1


def _mo8(v):
    return pl.multiple_of(v, 8)


def _pairs_body(x_hbm, x1_hbm, x2_hbm, x_v, o1_v, o2_v, sem_x, sem_o):
    wid = lax.axis_index("s") * 2 + lax.axis_index("c")
    cp_x = pltpu.make_async_copy(x_hbm, x_v.at[pl.ds(0, B)], sem_x)
    cp_x.start()
    cp_x.wait()

    def walk_rows(i0, pos0):
        def rbody(r, pos):
            i = i0 + r
            ln = jnp.int32(B - 1) - i
            n = (ln + 15) >> 4
            n4 = n >> 2
            nt = n & 3
            splat = plsc.load_gather(x_v, [jnp.full((16,), i, jnp.int32)])
            j0 = i + 1

            def g4(t, c):
                q = pos + t * 64
                jq = j0 + t * 64
                for u in range(4):
                    o1_v[pl.ds(q + u * 16, 16)] = splat
                    o2_v[pl.ds(q + u * 16, 16)] = x_v[pl.ds(jq + u * 16, 16)]
                return c

            lax.fori_loop(0, n4, g4, 0)
            q0 = pos + (n4 << 6)
            jt = j0 + (n4 << 6)

            def g1(t, c):
                o1_v[pl.ds(q0 + t * 16, 16)] = splat
                o2_v[pl.ds(q0 + t * 16, 16)] = x_v[pl.ds(jt + t * 16, 16)]
                return c

            lax.fori_loop(0, nt, g1, 0)
            return pos + ln

        return lax.fori_loop(0, 16, rbody, pos0)

    def dma_block(boff, obase, a):
        c2 = a >> 3
        c1 = a & 7

        def start(src_off, dst_off, sz, out_v, out_hbm):
            pltpu.make_async_copy(
                out_v.at[pl.ds(_mo8(src_off), sz)],
                out_hbm.at[pl.ds(_mo8(dst_off), sz)],
                sem_o,
            ).start()

        def d2(t, c):
            s = t * 2048
            start(boff + s, obase + s, 2048, o1_v, x1_hbm)
            start(boff + s, obase + s, 2048, o2_v, x2_hbm)
            return c

        lax.fori_loop(0, c2, d2, 0)
        s1 = c2 << 11

        def d1(t, c):
            s = s1 + t * 256
            start(boff + s, obase + s, 256, o1_v, x1_hbm)
            start(boff + s, obase + s, 256, o2_v, x2_hbm)
            return c

        lax.fori_loop(0, c1, d1, 0)
        st = a << 8
        start(boff + st, obase + st, 120, o1_v, x1_hbm)
        start(boff + st, obase + st, 120, o2_v, x2_hbm)

    iA0 = 16 * wid
    iB0 = jnp.int32(1008) - 16 * wid
    aA = jnp.int32(63) - wid
    la = 16248 - 256 * wid
    oa = (8 * wid) * (TWO_B_M1 - 16 * wid)
    m = jnp.int32(63) - wid
    ob = (8 * m) * (TWO_B_M1 - 16 * m)

    walk_rows(iA0, 0)
    dma_block(0, oa, aA)
    walk_rows(iB0, la + 16)
    dma_block(la + 16, ob, wid)

    pltpu.make_async_copy(
        x1_hbm.at[pl.ds(0, CHUNK)], o1_v.at[pl.ds(0, CHUNK)], sem_o
    ).wait()
    pltpu.make_async_copy(
        x2_hbm.at[pl.ds(0, CHUNK)], o2_v.at[pl.ds(0, CHUNK)], sem_o
    ).wait()


@functools.cache
def _build():
    return functools.partial(
        pl.kernel,
        out_type=(
            jax.ShapeDtypeStruct((P,), jnp.float32),
            jax.ShapeDtypeStruct((P,), jnp.float32),
        ),
        mesh=plsc.VectorSubcoreMesh(
            core_axis_name="c", subcore_axis_name="s", num_cores=2, num_subcores=16
        ),
        scratch_types=[
            pltpu.VMEM((XPAD,), jnp.float32),
            pltpu.VMEM((BUFN,), jnp.float32),
            pltpu.VMEM((BUFN,), jnp.float32),
            pltpu.SemaphoreType.DMA,
            pltpu.SemaphoreType.DMA,
        ],
        compiler_params=pltpu.CompilerParams(
            needs_layout_passes=False, disable_bounds_checks=True
        ),
    )(_pairs_body)


def kernel(x):
    return _build()(x)


# speedup vs baseline: 1.0628x; 1.0004x over previous
import functools

import jax
import jax.numpy as jnp
from jax import lax
from jax.experimental import pallas as pl
from jax.experimental.pallas import tpu as pltpu
from jax.experimental.pallas import tpu_sc as plsc

B = 1024
P = B * (B - 1) // 2
NW = 32
CHUNK = P // NW
BUFN = CHUNK + 32
XPAD = 1040
TWO_B_M1 = 2 * B - 1


def _mo8(v):
    return pl.multiple_of(v, 8)


def _pairs_body(x_hbm, x1_hbm, x2_hbm, x_v, o1_v, o2_v, sem_x, sem_o):
    wid = lax.axis_index("s") * 2 + lax.axis_index("c")
    cp_x = pltpu.make_async_copy(x_hbm, x_v.at[pl.ds(0, B)], sem_x)
    cp_x.start()
    cp_x.wait()

    def walk_rows(i0, pos0):
        def rbody(r, pos):
            i = i0 + r
            ln = jnp.int32(B - 1) - i
            n = (ln + 15) >> 4
            n4 = n >> 2
            nt = n & 3
            splat = plsc.load_gather(x_v, [jnp.full((16,), i, jnp.int32)])
            j0 = i + 1

            def g4(t, c):
                q = pos + t * 64
                jq = j0 + t * 64
                for u in range(4):
                    o1_v[pl.ds(q + u * 16, 16)] = splat
                    o2_v[pl.ds(q + u * 16, 16)] = x_v[pl.ds(jq + u * 16, 16)]
                return c

            lax.fori_loop(0, n4, g4, 0)
            q0 = pos + (n4 << 6)
            jt = j0 + (n4 << 6)

            def g1(t, c):
                o1_v[pl.ds(q0 + t * 16, 16)] = splat
                o2_v[pl.ds(q0 + t * 16, 16)] = x_v[pl.ds(jt + t * 16, 16)]
                return c

            lax.fori_loop(0, nt, g1, 0)
            return pos + ln

        return lax.fori_loop(0, 16, rbody, pos0)

    def dma_block(boff, obase, a):
        c2 = a >> 3
        c1 = a & 7

        def start(src_off, dst_off, sz, out_v, out_hbm):
            pltpu.make_async_copy(
                out_v.at[pl.ds(_mo8(src_off), sz)],
                out_hbm.at[pl.ds(_mo8(dst_off), sz)],
                sem_o,
            ).start()

        def d2(t, c):
            s = t * 2048
            start(boff + s, obase + s, 2048, o1_v, x1_hbm)
            start(boff + s, obase + s, 2048, o2_v, x2_hbm)
            return c

        lax.fori_loop(0, c2, d2, 0)
        s1 = c2 << 11

        def d1(t, c):
            s = s1 + t * 256
            start(boff + s, obase + s, 256, o1_v, x1_hbm)
            start(boff + s, obase + s, 256, o2_v, x2_hbm)
            return c

        lax.fori_loop(0, c1, d1, 0)
        st = a << 8
        start(boff + st, obase + st, 120, o1_v, x1_hbm)
        start(boff + st, obase + st, 120, o2_v, x2_hbm)

    iA0 = 16 * wid
    iB0 = jnp.int32(1008) - 16 * wid
    aA = jnp.int32(63) - wid
    la = 16248 - 256 * wid
    oa = (8 * wid) * (TWO_B_M1 - 16 * wid)
    m = jnp.int32(63) - wid
    ob = (8 * m) * (TWO_B_M1 - 16 * m)

    walk_rows(iA0, 0)
    dma_block(0, oa, aA)
    walk_rows(iB0, la + 16)
    dma_block(la + 16, ob, wid)

    pltpu.make_async_copy(
        x1_hbm.at[pl.ds(0, CHUNK)], o1_v.at[pl.ds(0, CHUNK)], sem_o
    ).wait()
    pltpu.make_async_copy(
        x2_hbm.at[pl.ds(0, CHUNK)], o2_v.at[pl.ds(0, CHUNK)], sem_o
    ).wait()


@functools.cache
def _build():
    return functools.partial(
        pl.kernel,
        out_type=(
            jax.ShapeDtypeStruct((P,), jnp.float32),
            jax.ShapeDtypeStruct((P,), jnp.float32),
        ),
        mesh=plsc.VectorSubcoreMesh(
            core_axis_name="c", subcore_axis_name="s", num_cores=2, num_subcores=16
        ),
        scratch_types=[
            pltpu.VMEM((XPAD,), jnp.float32),
            pltpu.VMEM((BUFN,), jnp.float32),
            pltpu.VMEM((BUFN,), jnp.float32),
            pltpu.SemaphoreType.DMA,
            pltpu.SemaphoreType.DMA,
        ],
        compiler_params=pltpu.CompilerParams(
            needs_layout_passes=False, disable_bounds_checks=True, skip_device_barrier=True
        ),
    )(_pairs_body)


def kernel(x):
    return _build()(x)


# speedup vs baseline: 1.0628x; 1.0001x over previous
import functools

import jax
import jax.numpy as jnp
from jax import lax
from jax.experimental import pallas as pl
from jax.experimental.pallas import tpu as pltpu
from jax.experimental.pallas import tpu_sc as plsc

B = 1024
P = B * (B - 1) // 2
NW = 32
CHUNK = P // NW
BUFN = CHUNK + 32
XPAD = 1040
TWO_B_M1 = 2 * B - 1


def _mo8(v):
    return pl.multiple_of(v, 8)


def _pairs_body(x_hbm, x1_hbm, x2_hbm, x_v, o1_v, o2_v, sem_x, sem_o):
    wid = lax.axis_index("s") * 2 + lax.axis_index("c")
    cp_x = pltpu.make_async_copy(x_hbm, x_v.at[pl.ds(0, B)], sem_x)
    cp_x.start()
    cp_x.wait()

    def walk_rows(i0, pos0):
        def rbody(r, pos):
            i = i0 + r
            ln = jnp.int32(B - 1) - i
            n = (ln + 15) >> 4
            n4 = n >> 2
            nt = n & 3
            splat = plsc.load_gather(x_v, [jnp.full((16,), i, jnp.int32)])
            j0 = i + 1

            def g4(t, c):
                q = pos + t * 64
                jq = j0 + t * 64
                for u in range(4):
                    o1_v[pl.ds(q + u * 16, 16)] = splat
                    o2_v[pl.ds(q + u * 16, 16)] = x_v[pl.ds(jq + u * 16, 16)]
                return c

            lax.fori_loop(0, n4, g4, 0)
            q0 = pos + (n4 << 6)
            jt = j0 + (n4 << 6)

            def g1(t, c):
                o1_v[pl.ds(q0 + t * 16, 16)] = splat
                o2_v[pl.ds(q0 + t * 16, 16)] = x_v[pl.ds(jt + t * 16, 16)]
                return c

            lax.fori_loop(0, nt, g1, 0)
            return pos + ln

        return lax.fori_loop(0, 16, rbody, pos0)

    def dma_block(boff, obase, a):
        c2 = a >> 3
        c1 = a & 7

        def start(src_off, dst_off, sz, out_v, out_hbm):
            pltpu.make_async_copy(
                out_v.at[pl.ds(_mo8(src_off), sz)],
                out_hbm.at[pl.ds(_mo8(dst_off), sz)],
                sem_o,
            ).start()

        def d2(t, c):
            s = t * 2048
            start(boff + s, obase + s, 2048, o1_v, x1_hbm)
            start(boff + s, obase + s, 2048, o2_v, x2_hbm)
            return c

        lax.fori_loop(0, c2, d2, 0)
        s1 = c2 << 11

        def d1(t, c):
            s = s1 + t * 256
            start(boff + s, obase + s, 256, o1_v, x1_hbm)
            start(boff + s, obase + s, 256, o2_v, x2_hbm)
            return c

        lax.fori_loop(0, c1, d1, 0)
        st = a << 8
        start(boff + st, obase + st, 120, o1_v, x1_hbm)
        start(boff + st, obase + st, 120, o2_v, x2_hbm)

    iA0 = 16 * wid
    iB0 = jnp.int32(1008) - 16 * wid
    aA = jnp.int32(63) - wid
    la = 16248 - 256 * wid
    oa = (8 * wid) * (TWO_B_M1 - 16 * wid)
    m = jnp.int32(63) - wid
    ob = (8 * m) * (TWO_B_M1 - 16 * m)

    walk_rows(iA0, 0)
    dma_block(0, oa, aA)
    walk_rows(iB0, la + 16)
    dma_block(la + 16, ob, wid)

    pltpu.make_async_copy(
        x1_hbm.at[pl.ds(0, CHUNK)], o1_v.at[pl.ds(0, CHUNK)], sem_o
    ).wait()
    pltpu.make_async_copy(
        x2_hbm.at[pl.ds(0, CHUNK)], o2_v.at[pl.ds(0, CHUNK)], sem_o
    ).wait()


@functools.cache
def _build():
    return functools.partial(
        pl.kernel,
        out_type=(
            jax.ShapeDtypeStruct((P,), jnp.float32),
            jax.ShapeDtypeStruct((P,), jnp.float32),
        ),
        mesh=plsc.VectorSubcoreMesh(
            core_axis_name="c", subcore_axis_name="s", num_cores=2, num_subcores=16
        ),
        scratch_types=[
            pltpu.VMEM((XPAD,), jnp.float32),
            pltpu.VMEM((BUFN,), jnp.float32),
            pltpu.VMEM((BUFN,), jnp.float32),
            pltpu.SemaphoreType.DMA,
            pltpu.SemaphoreType.DMA,
        ],
        compiler_params=pltpu.CompilerParams(
            needs_layout_passes=False, disable_bounds_checks=True
        ),
    )(_pairs_body)


def kernel(x):
    return _build()(x)
